# baseline (speedup 1.0000x reference)
import functools

import jax
import jax.numpy as jnp
from jax import lax
from jax.experimental import pallas as pl
from jax.experimental.pallas import tpu as pltpu
from jax.experimental.pallas import tpu_sc as plsc

B = 32
H = 128
KV_LORA = 512
ROPE = 64
D = KV_LORA + ROPE
KV_LEN = 8192
TOPK = 2048
SCALE = 1.0 / (192.0 ** 0.5)

_NC = 2
_NS = 16
_NW = _NC * _NS
_L = 16


def _hist_body(idx_hbm, cnt_hbm, idx_v, hist_v):
    wid = lax.axis_index("s") * _NC + lax.axis_index("c")
    pltpu.sync_copy(idx_hbm.at[wid], idx_v)

    zeros = jnp.zeros((_L,), jnp.float32)

    def zbody(i, carry):
        hist_v[pl.ds(i * _L, _L)] = zeros
        return carry

    lax.fori_loop(0, KV_LEN // _L, zbody, 0)

    ones = jnp.ones((_L,), jnp.float32)

    def body(i, carry):
        iv = idx_v[pl.ds(i * _L, _L)]
        plsc.addupdate_scatter(hist_v, [iv], ones)
        return carry

    lax.fori_loop(0, TOPK // _L, body, 0)
    pltpu.sync_copy(hist_v, cnt_hbm.at[wid])


@functools.cache
def _sc_hist():
    return pl.kernel(
        _hist_body,
        mesh=plsc.VectorSubcoreMesh(core_axis_name="c", subcore_axis_name="s"),
        out_type=jax.ShapeDtypeStruct((B, KV_LEN), jnp.float32),
        scratch_types=[
            pltpu.VMEM((TOPK,), jnp.int32),
            pltpu.VMEM((KV_LEN,), jnp.float32),
        ],
        compiler_params=pltpu.CompilerParams(needs_layout_passes=False),
    )


def _attn_kernel(q_ref, kva_ref, cnt_ref, o_ref):
    q = q_ref[0].astype(jnp.bfloat16)
    kv = kva_ref[0].astype(jnp.bfloat16)
    cnt = cnt_ref[0, 0]
    lc = jnp.where(cnt > 0.0, jnp.log(cnt), -1e30)
    logits = lax.dot_general(
        q, kv, (((1,), (1,)), ((), ())), preferred_element_type=jnp.float32
    ) * SCALE + lc[None, :]
    m = jnp.max(logits, axis=-1, keepdims=True)
    p = jnp.exp(logits - m)
    denom = jnp.sum(p, axis=-1, keepdims=True)
    o = lax.dot_general(
        p.astype(jnp.bfloat16), kv[:, :KV_LORA], (((1,), (0,)), ((), ())),
        preferred_element_type=jnp.float32,
    )
    o_ref[0] = o / denom


def _dummy_body(kv_hbm, out_hbm, buf_v):
    wid = lax.axis_index("s") * _NC + lax.axis_index("c")
    def body(i, carry):
        pltpu.sync_copy(kv_hbm.at[wid, pl.ds(i * 128, 128)], buf_v)
        pltpu.sync_copy(buf_v, out_hbm.at[wid])
        return carry
    lax.fori_loop(0, 16, body, 0)


@functools.cache
def _sc_dummy():
    return pl.kernel(
        _dummy_body,
        mesh=plsc.VectorSubcoreMesh(core_axis_name="c", subcore_axis_name="s"),
        out_type=jax.ShapeDtypeStruct((B, 128, D), jnp.float32),
        scratch_types=[pltpu.VMEM((128, D), jnp.float32)],
        compiler_params=pltpu.CompilerParams(needs_layout_passes=False),
    )


def kernel(q, kv_cache, indices):
    dummy = _sc_dummy()(kv_cache)
    counts = _sc_hist()(indices.reshape(B, TOPK))

    qr = q.reshape(B, H, D)
    out = pl.pallas_call(
        _attn_kernel,
        grid=(B,),
        in_specs=[
            pl.BlockSpec((1, H, D), lambda b: (b, 0, 0)),
            pl.BlockSpec((1, KV_LEN, D), lambda b: (b, 0, 0)),
            pl.BlockSpec((1, 1, KV_LEN), lambda b: (b, 0, 0)),
        ],
        out_specs=pl.BlockSpec((1, H, KV_LORA), lambda b: (b, 0, 0)),
        out_shape=jax.ShapeDtypeStruct((B, H, KV_LORA), jnp.float32),
    )(qr, kv_cache, counts.reshape(B, 1, KV_LEN))
    out = out + jnp.minimum(jnp.abs(jnp.min(dummy)), 0.0)
    return out.reshape(B, 1, H, KV_LORA)


# speedup vs baseline: 1.1744x; 1.1744x over previous
import functools

import jax
import jax.numpy as jnp
from jax import lax
from jax.experimental import pallas as pl
from jax.experimental.pallas import tpu as pltpu
from jax.experimental.pallas import tpu_sc as plsc

B = 32
H = 128
KV_LORA = 512
ROPE = 64
D = KV_LORA + ROPE
KV_LEN = 8192
TOPK = 2048
SCALE = 1.0 / (192.0 ** 0.5)

_NC = 2
_NS = 16
_NW = _NC * _NS
_L = 16


def _hist_body(idx_hbm, cnt_hbm, idx_v, hist_v):
    wid = lax.axis_index("s") * _NC + lax.axis_index("c")
    pltpu.sync_copy(idx_hbm.at[wid], idx_v)

    zeros = jnp.zeros((_L,), jnp.float32)

    def zbody(i, carry):
        hist_v[pl.ds(i * _L, _L)] = zeros
        return carry

    lax.fori_loop(0, KV_LEN // _L, zbody, 0)

    ones = jnp.ones((_L,), jnp.float32)

    def body(i, carry):
        iv = idx_v[pl.ds(i * _L, _L)]
        plsc.addupdate_scatter(hist_v, [iv], ones)
        return carry

    lax.fori_loop(0, TOPK // _L, body, 0)
    pltpu.sync_copy(hist_v, cnt_hbm.at[wid])


@functools.cache
def _sc_hist():
    return pl.kernel(
        _hist_body,
        mesh=plsc.VectorSubcoreMesh(core_axis_name="c", subcore_axis_name="s"),
        out_type=jax.ShapeDtypeStruct((B, KV_LEN), jnp.float32),
        scratch_types=[
            pltpu.VMEM((TOPK,), jnp.int32),
            pltpu.VMEM((KV_LEN,), jnp.float32),
        ],
        compiler_params=pltpu.CompilerParams(
            needs_layout_passes=False, use_tc_tiling_on_sc=True
        ),
    )


def _attn_kernel(q_ref, kva_ref, cnt_ref, o_ref):
    q = q_ref[0].astype(jnp.bfloat16)
    kv = kva_ref[0].astype(jnp.bfloat16)
    cnt = cnt_ref[0, 0]
    lc = jnp.where(cnt > 0.0, jnp.log(cnt), -1e30)
    logits = lax.dot_general(
        q, kv, (((1,), (1,)), ((), ())), preferred_element_type=jnp.float32
    ) * SCALE + lc[None, :]
    m = jnp.max(logits, axis=-1, keepdims=True)
    p = jnp.exp(logits - m)
    denom = jnp.sum(p, axis=-1, keepdims=True)
    o = lax.dot_general(
        p.astype(jnp.bfloat16), kv[:, :KV_LORA], (((1,), (0,)), ((), ())),
        preferred_element_type=jnp.float32,
    )
    o_ref[0] = o / denom


def kernel(q, kv_cache, indices):
    counts = _sc_hist()(indices.reshape(B, TOPK))

    qr = q.reshape(B, H, D)
    out = pl.pallas_call(
        _attn_kernel,
        grid=(B,),
        in_specs=[
            pl.BlockSpec((1, H, D), lambda b: (b, 0, 0)),
            pl.BlockSpec((1, KV_LEN, D), lambda b: (b, 0, 0)),
            pl.BlockSpec((1, 1, KV_LEN), lambda b: (b, 0, 0)),
        ],
        out_specs=pl.BlockSpec((1, H, KV_LORA), lambda b: (b, 0, 0)),
        out_shape=jax.ShapeDtypeStruct((B, H, KV_LORA), jnp.float32),
    )(qr, kv_cache, counts.reshape(B, 1, KV_LEN))
    return out.reshape(B, 1, H, KV_LORA)


# speedup vs baseline: 1.1762x; 1.0015x over previous
import functools

import jax
import jax.numpy as jnp
from jax import lax
from jax.experimental import pallas as pl
from jax.experimental.pallas import tpu as pltpu
from jax.experimental.pallas import tpu_sc as plsc

B = 32
H = 128
KV_LORA = 512
ROPE = 64
D = KV_LORA + ROPE
KV_LEN = 8192
TOPK = 2048
SCALE = 1.0 / (192.0 ** 0.5)

_NC = 2
_NS = 16
_NW = _NC * _NS
_L = 16


def _hist_body(idx_hbm, cnt_hbm, idx_v, hist_v):
    wid = lax.axis_index("s") * _NC + lax.axis_index("c")
    pltpu.sync_copy(idx_hbm.at[wid, 0], idx_v)

    zeros = jnp.zeros((_L,), jnp.float32)

    def zbody(i, carry):
        hist_v[pl.ds(i * _L, _L)] = zeros
        return carry

    lax.fori_loop(0, KV_LEN // _L, zbody, 0)

    ones = jnp.ones((_L,), jnp.float32)

    def body(i, carry):
        iv = idx_v[pl.ds(i * _L, _L)]
        plsc.addupdate_scatter(hist_v, [iv], ones)
        return carry

    lax.fori_loop(0, TOPK // _L, body, 0)
    pltpu.sync_copy(hist_v, cnt_hbm.at[wid])


@functools.cache
def _sc_hist():
    return pl.kernel(
        _hist_body,
        mesh=plsc.VectorSubcoreMesh(core_axis_name="c", subcore_axis_name="s"),
        out_type=jax.ShapeDtypeStruct((B, KV_LEN), jnp.float32),
        scratch_types=[
            pltpu.VMEM((TOPK,), jnp.int32),
            pltpu.VMEM((KV_LEN,), jnp.float32),
        ],
        compiler_params=pltpu.CompilerParams(
            needs_layout_passes=False, use_tc_tiling_on_sc=True
        ),
    )


def _attn_kernel(q_ref, kva_ref, cnt_ref, o_ref):
    q = q_ref[0].astype(jnp.bfloat16)
    kv = kva_ref[0].astype(jnp.bfloat16)
    cnt = cnt_ref[0, 0]
    lc = jnp.where(cnt > 0.0, jnp.log(cnt), -1e30)
    logits = lax.dot_general(
        q, kv, (((1,), (1,)), ((), ())), preferred_element_type=jnp.float32
    ) * SCALE + lc[None, :]
    m = jnp.max(logits, axis=-1, keepdims=True)
    p = jnp.exp(logits - m)
    denom = jnp.sum(p, axis=-1, keepdims=True)
    o = lax.dot_general(
        p.astype(jnp.bfloat16), kv[:, :KV_LORA], (((1,), (0,)), ((), ())),
        preferred_element_type=jnp.float32,
    )
    o_ref[0] = o / denom


def kernel(q, kv_cache, indices):
    counts = _sc_hist()(indices)

    qr = q.reshape(B, H, D)
    out = pl.pallas_call(
        _attn_kernel,
        grid=(B,),
        in_specs=[
            pl.BlockSpec((1, H, D), lambda b: (b, 0, 0)),
            pl.BlockSpec((1, KV_LEN, D), lambda b: (b, 0, 0)),
            pl.BlockSpec((1, 1, KV_LEN), lambda b: (b, 0, 0)),
        ],
        out_specs=pl.BlockSpec((1, H, KV_LORA), lambda b: (b, 0, 0)),
        out_shape=jax.ShapeDtypeStruct((B, H, KV_LORA), jnp.float32),
    )(qr, kv_cache, counts.reshape(B, 1, KV_LEN))
    return out.reshape(B, 1, H, KV_LORA)
